# baseline (speedup 1.0000x reference)
import jax
import jax.numpy as jnp
from jax import lax
from jax.experimental import pallas as pl
from jax.experimental.pallas import tpu as pltpu
from jax.experimental.pallas import tpu_sc as plsc

B = 16384
D = 64
E = 1000000

_info = plsc.get_sparse_core_info()
NC = _info.num_cores
NS = _info.num_subcores
L = _info.num_lanes
NW = NC * NS

TCOLS = 7813
PARTIAL_COL = 7812
CPW = 248
CC = 4
NCH = CPW // CC
WCAP = 1040
NVR = 1024
BROWS = B + L


def _popcount(mask):
    return plsc.all_reduce_population_count(mask)[0]


def _build_worklist(idxb, entl, posl, cbase):
    def body(k, cnt):
        e = idxb[pl.ds(k * L, L)]
        ec = jax.lax.shift_right_logical(e, 7)
        m = (ec >= cbase) & (ec < cbase + CPW)
        plsc.store_compressed(entl.at[pl.ds(cnt, L)], e, mask=m)
        pos = k * L + lax.iota(jnp.int32, L)
        plsc.store_compressed(posl.at[pl.ds(cnt, L)], pos, mask=m)
        return cnt + _popcount(m)
    return lax.fori_loop(0, NVR, body, jnp.int32(0))


def _chunk_dma(table, tail, slab, sem, cstart, start=True):
    @pl.when(cstart <= PARTIAL_COL - CC)
    def _():
        for k in range(CC):
            cp = pltpu.make_async_copy(
                table.at[pl.ds(0, D), pl.ds((cstart + k) * 128, 128)],
                slab.at[pl.ds(k * D, D), :], sem)
            cp.start() if start else cp.wait()

    @pl.when(cstart == PARTIAL_COL)
    def _():
        cp = pltpu.make_async_copy(tail, slab.at[pl.ds(0, D), :], sem)
        cp.start() if start else cp.wait()


def _process_chunk(slab, cstart, lists):
    new_sctrs = []
    for (entl, posl, cnt, dstrows, stg, spos, ssem, sctr) in lists:
        cel_m = jnp.full((L,), 0, jnp.int32)
        del cel_m

        def ext_body(k, carry):
            mcnt, = carry
            e = entl[pl.ds(k * L, L)]
            p = posl[pl.ds(k * L, L)]
            valid = (k * L + lax.iota(jnp.int32, L)) < cnt
            ecc = jax.lax.shift_right_logical(e, 7) - cstart
            m = valid & (ecc >= 0) & (ecc < CC)
            plsc.store_compressed(_process_chunk.ce.at[pl.ds(mcnt, L)], e, mask=m)
            plsc.store_compressed(_process_chunk.cp.at[pl.ds(mcnt, L)], p, mask=m)
            return (mcnt + _popcount(m),)

        nvreg = jax.lax.shift_right_logical(cnt + L - 1, 4)
        (mcnt,) = lax.fori_loop(0, nvreg, ext_body, (jnp.int32(0),))

        def grp_body(gi, sc):
            @pl.when(sc >= 1)
            def _():
                pltpu.make_async_copy(stg, dstrows.at[spos], ssem).wait()
            lanes = lax.iota(jnp.int32, L)
            gvalid = (gi * L + lanes) < mcnt
            ev = jnp.where(gvalid, _process_chunk.ce[pl.ds(gi * L, L)],
                           cstart * 128)
            pv = jnp.where(gvalid, _process_chunk.cp[pl.ds(gi * L, L)], B)
            for jj in range(L):
                e_s = ev[jj]
                colin = jax.lax.shift_right_logical(e_s, 7) - cstart
                lane = jnp.bitwise_and(e_s, 127)
                lanev = jnp.full((L,), lane, jnp.int32)
                for g in range(D // L):
                    rowv = colin * D + g * L + lax.iota(jnp.int32, L)
                    v = plsc.load_gather(slab, [rowv, lanev])
                    stg[jj, pl.ds(g * L, L)] = v
            spos[...] = pv
            pltpu.make_async_copy(stg, dstrows.at[spos], ssem).start()
            return sc + 1
        ngrp = jax.lax.shift_right_logical(mcnt + L - 1, 4)
        new_sctrs.append(lax.fori_loop(0, ngrp, grp_body, sctr))
    return new_sctrs


def _table_scan(table, tail, slabA, slabB, dsem, cbase, lists):
    dsemA, dsemB = dsem

    def super_body(k, sctrs):
        cA = cbase + (2 * k) * CC
        cB = cbase + (2 * k + 1) * CC
        _chunk_dma(table, tail, slabB, dsemB, cB, start=True)
        _chunk_dma(table, tail, slabA, dsemA, cA, start=False)
        sctrs = _process_chunk(slabA, cA, _bind(lists, sctrs))

        @pl.when(2 * k + 2 < NCH)
        def _():
            _chunk_dma(table, tail, slabA, dsemA, cbase + (2 * k + 2) * CC,
                       start=True)
        _chunk_dma(table, tail, slabB, dsemB, cB, start=False)
        sctrs = _process_chunk(slabB, cB, _bind(lists, sctrs))
        return tuple(sctrs)

    _chunk_dma(table, tail, slabA, dsemA, cbase, start=True)
    return lax.fori_loop(0, NCH // 2, super_body,
                         tuple(jnp.int32(0) for _ in lists))


def _bind(lists, sctrs):
    return [tuple(l) + (s,) for l, s in zip(lists, sctrs)]


def _gather_body(user, itemi, itemj, eu_t, ei_t, tail_u, tail_i,
                 urows, virows, vjrows,
                 idxb, ent_u, pos_u, ent_i, pos_i, ent_j, pos_j,
                 ce, cp, slabA, slabB,
                 stg_u, stg_i, stg_j, spos_u, spos_i, spos_j,
                 dsemA, dsemB, ssem_u, ssem_i, ssem_j):
    _process_chunk.ce = ce
    _process_chunk.cp = cp
    wid = lax.axis_index("s") * NC + lax.axis_index("c")
    cbase = wid * CPW

    pltpu.sync_copy(user, idxb)
    cnt_u = _build_worklist(idxb, ent_u, pos_u, cbase)
    pltpu.sync_copy(itemi, idxb)
    cnt_i = _build_worklist(idxb, ent_i, pos_i, cbase)
    pltpu.sync_copy(itemj, idxb)
    cnt_j = _build_worklist(idxb, ent_j, pos_j, cbase)

    (sc_u,) = _table_scan(eu_t, tail_u, slabA, slabB, (dsemA, dsemB), cbase,
                          [(ent_u, pos_u, cnt_u, urows, stg_u, spos_u, ssem_u)])
    (sc_i, sc_j) = _table_scan(
        ei_t, tail_i, slabA, slabB, (dsemA, dsemB), cbase,
        [(ent_i, pos_i, cnt_i, virows, stg_i, spos_i, ssem_i),
         (ent_j, pos_j, cnt_j, vjrows, stg_j, spos_j, ssem_j)])

    for sc, stg, spos, dst, ssem in (
            (sc_u, stg_u, spos_u, urows, ssem_u),
            (sc_i, stg_i, spos_i, virows, ssem_i),
            (sc_j, stg_j, spos_j, vjrows, ssem_j)):
        @pl.when(sc >= 1)
        def _():
            pltpu.make_async_copy(stg, dst.at[spos], ssem).wait()


def _dot_body(u_ref, vi_ref, vj_ref, pi_ref, pj_ref):
    u = u_ref[:, :D]
    pi_ref[...] = jnp.sum(u * vi_ref[:, :D], axis=1)
    pj_ref[...] = jnp.sum(u * vj_ref[:, :D], axis=1)


def kernel(user, item_i, item_j, embed_user, embed_item):
    eu_t = embed_user.T
    ei_t = embed_item.T
    tail_u = jnp.pad(embed_user[PARTIAL_COL * 128:].T, ((0, 0), (0, D)))
    tail_i = jnp.pad(embed_item[PARTIAL_COL * 128:].T, ((0, 0), (0, D)))
    mesh = plsc.VectorSubcoreMesh(core_axis_name="c", subcore_axis_name="s")
    gather = pl.kernel(
        _gather_body,
        mesh=mesh,
        out_type=(jax.ShapeDtypeStruct((BROWS, 128), jnp.float32),
                  jax.ShapeDtypeStruct((BROWS, 128), jnp.float32),
                  jax.ShapeDtypeStruct((BROWS, 128), jnp.float32)),
        scratch_types=[
            pltpu.VMEM((B,), jnp.int32),
            pltpu.VMEM((WCAP,), jnp.int32),
            pltpu.VMEM((WCAP,), jnp.int32),
            pltpu.VMEM((WCAP,), jnp.int32),
            pltpu.VMEM((WCAP,), jnp.int32),
            pltpu.VMEM((WCAP,), jnp.int32),
            pltpu.VMEM((WCAP,), jnp.int32),
            pltpu.VMEM((WCAP,), jnp.int32),
            pltpu.VMEM((WCAP,), jnp.int32),
            pltpu.VMEM((CC * D, 128), jnp.float32),
            pltpu.VMEM((CC * D, 128), jnp.float32),
            pltpu.VMEM((L, 128), jnp.float32),
            pltpu.VMEM((L, 128), jnp.float32),
            pltpu.VMEM((L, 128), jnp.float32),
            pltpu.VMEM((L,), jnp.int32),
            pltpu.VMEM((L,), jnp.int32),
            pltpu.VMEM((L,), jnp.int32),
            pltpu.SemaphoreType.DMA,
            pltpu.SemaphoreType.DMA,
            pltpu.SemaphoreType.DMA,
            pltpu.SemaphoreType.DMA,
            pltpu.SemaphoreType.DMA,
        ],
        compiler_params=pltpu.CompilerParams(needs_layout_passes=False,
                                             use_tc_tiling_on_sc=True),
    )
    urows, virows, vjrows = gather(user, item_i, item_j, eu_t, ei_t,
                                   tail_u, tail_i)

    dot = pl.pallas_call(
        _dot_body,
        grid=(NW,),
        in_specs=[pl.BlockSpec((B // NW, 128), lambda i: (i, 0))] * 3,
        out_specs=[pl.BlockSpec((B // NW,), lambda i: (i,))] * 2,
        out_shape=(jax.ShapeDtypeStruct((B,), jnp.float32),
                   jax.ShapeDtypeStruct((B,), jnp.float32)),
    )
    return dot(urows, virows, vjrows)


# speedup vs baseline: 5.4595x; 5.4595x over previous
import jax
import jax.numpy as jnp
from jax import lax
from jax.experimental import pallas as pl
from jax.experimental.pallas import tpu as pltpu
from jax.experimental.pallas import tpu_sc as plsc

B = 16384
D = 64
E = 1000000

_info = plsc.get_sparse_core_info()
NC = _info.num_cores
NS = _info.num_subcores
L = _info.num_lanes
NW = NC * NS

TCOLS = 7813
PARTIAL_COL = 7812
CPW = 252
CC = 6
NCH = CPW // CC
WCAP = 1040
NVR = 1024
BROWS = B + L


def _popcount(mask):
    return plsc.all_reduce_population_count(mask)[0]


def _build_worklist(idxb, entl, posl, cbase):
    def body(k, cnt):
        e = idxb[pl.ds(k * L, L)]
        ec = jax.lax.shift_right_logical(e, 7)
        m = (ec >= cbase) & (ec < cbase + CPW)
        plsc.store_compressed(entl.at[pl.ds(cnt, L)], e, mask=m)
        pos = k * L + lax.iota(jnp.int32, L)
        plsc.store_compressed(posl.at[pl.ds(cnt, L)], pos, mask=m)
        return cnt + _popcount(m)
    return lax.fori_loop(0, NVR, body, jnp.int32(0))


def _chunk_dma(table, tail, slab, sem, cstart, start=True):
    @pl.when(cstart <= PARTIAL_COL - CC)
    def _():
        for di in range(D // 8):
            cp = pltpu.make_async_copy(
                table.at[pl.ds(di * 8, 8), pl.ds(cstart * 128, CC * 128)],
                slab.at[pl.ds(di * 8, 8), :], sem)
            cp.start() if start else cp.wait()

    @pl.when(cstart == PARTIAL_COL)
    def _():
        cp = pltpu.make_async_copy(tail, slab.at[:, pl.ds(0, 128)], sem)
        cp.start() if start else cp.wait()


def _process_chunk(slab, cstart, lists):
    new_sctrs = []
    for (entl, posl, cnt, dstrows, stg, spos, ssem, (sctr, nstg)) in lists:
        cel_m = jnp.full((L,), 0, jnp.int32)
        del cel_m

        def ext_body(k, carry):
            mcnt, = carry
            e = entl[pl.ds(k * L, L)]
            p = posl[pl.ds(k * L, L)]
            valid = (k * L + lax.iota(jnp.int32, L)) < cnt
            ecc = jax.lax.shift_right_logical(e, 7) - cstart
            m = valid & (ecc >= 0) & (ecc < CC)
            plsc.store_compressed(_process_chunk.ce.at[pl.ds(mcnt, L)], e, mask=m)
            plsc.store_compressed(_process_chunk.cp.at[pl.ds(mcnt, L)], p, mask=m)
            return (mcnt + _popcount(m),)

        nvreg = jax.lax.shift_right_logical(cnt + L - 1, 4)
        (mcnt,) = lax.fori_loop(0, nvreg, ext_body, (jnp.int32(0),))

        lanes = lax.iota(jnp.int32, L)

        def ent_body(n, carry):
            sc, nst = carry

            @pl.when((nst == 0) & (sc >= 1))
            def _():
                pltpu.make_async_copy(stg, dstrows.at[spos], ssem).wait()
            e_s = _process_chunk.ce[pl.ds(n, L)][0]
            p_s = _process_chunk.cp[pl.ds(n, L)][0]
            colin = jax.lax.shift_right_logical(e_s, 7) - cstart
            off = colin * 128 + jnp.bitwise_and(e_s, 127)
            offv = jnp.full((L,), off, jnp.int32)
            rowv = jnp.full((L,), nst, jnp.int32)
            for g in range(D // L):
                fv = g * L + lanes
                v = plsc.load_gather(slab, [fv, offv])
                plsc.store_scatter(stg, [rowv, fv], v)
            plsc.store_scatter(spos, [lanes], jnp.full((L,), p_s, jnp.int32),
                               mask=lanes == nst)

            @pl.when(nst == L - 1)
            def _():
                pltpu.make_async_copy(stg, dstrows.at[spos], ssem).start()
            full = nst == L - 1
            return (jnp.where(full, sc + 1, sc),
                    jnp.where(full, 0, nst + 1))

        new_sctrs.append(lax.fori_loop(0, mcnt, ent_body, (sctr, nstg)))
    return new_sctrs


def _table_scan(table, tail, slabA, slabB, dsem, cbase, lists, init_carries):
    dsemA, dsemB = dsem

    def super_body(k, sctrs):
        cA = cbase + (2 * k) * CC
        cB = cbase + (2 * k + 1) * CC
        _chunk_dma(table, tail, slabB, dsemB, cB, start=True)
        _chunk_dma(table, tail, slabA, dsemA, cA, start=False)
        sctrs = _process_chunk(slabA, cA, _bind(lists, sctrs))

        @pl.when(2 * k + 2 < NCH)
        def _():
            _chunk_dma(table, tail, slabA, dsemA, cbase + (2 * k + 2) * CC,
                       start=True)
        _chunk_dma(table, tail, slabB, dsemB, cB, start=False)
        sctrs = _process_chunk(slabB, cB, _bind(lists, sctrs))
        return tuple(sctrs)

    _chunk_dma(table, tail, slabA, dsemA, cbase, start=True)
    return lax.fori_loop(0, NCH // 2, super_body,
                         tuple(carries for carries in init_carries))


def _bind(lists, sctrs):
    return [tuple(l) + (s,) for l, s in zip(lists, sctrs)]


def _gather_body(user, itemi, itemj, eu_t, ei_t, tail_u, tail_i,
                 urows, virows, vjrows,
                 idxb, ent_u, pos_u, ent_i, pos_i, ent_j, pos_j,
                 ce, cp, slabA, slabB,
                 stg_u, stg_i, stg_j, spos_u, spos_i, spos_j,
                 dsemA, dsemB, ssem_u, ssem_i, ssem_j):
    _process_chunk.ce = ce
    _process_chunk.cp = cp
    wid = lax.axis_index("s") * NC + lax.axis_index("c")
    cbase = wid * CPW

    pltpu.sync_copy(user, idxb)
    cnt_u = _build_worklist(idxb, ent_u, pos_u, cbase)
    pltpu.sync_copy(itemi, idxb)
    cnt_i = _build_worklist(idxb, ent_i, pos_i, cbase)
    pltpu.sync_copy(itemj, idxb)
    cnt_j = _build_worklist(idxb, ent_j, pos_j, cbase)

    zz = (jnp.int32(0), jnp.int32(0))
    ((sc_u, nst_u),) = _table_scan(
        eu_t, tail_u, slabA, slabB, (dsemA, dsemB), cbase,
        [(ent_u, pos_u, cnt_u, urows, stg_u, spos_u, ssem_u)], (zz,))
    ((sc_i, nst_i), (sc_j, nst_j)) = _table_scan(
        ei_t, tail_i, slabA, slabB, (dsemA, dsemB), cbase,
        [(ent_i, pos_i, cnt_i, virows, stg_i, spos_i, ssem_i),
         (ent_j, pos_j, cnt_j, vjrows, stg_j, spos_j, ssem_j)], (zz, zz))

    lanes = lax.iota(jnp.int32, L)
    for sc, nst, stg, spos, dst, ssem in (
            (sc_u, nst_u, stg_u, spos_u, urows, ssem_u),
            (sc_i, nst_i, stg_i, spos_i, virows, ssem_i),
            (sc_j, nst_j, stg_j, spos_j, vjrows, ssem_j)):
        @pl.when(nst > 0)
        def _():
            plsc.store_scatter(spos, [lanes], jnp.full((L,), B, jnp.int32),
                               mask=lanes >= nst)
            pltpu.make_async_copy(stg, dst.at[spos], ssem).start()
            pltpu.make_async_copy(stg, dst.at[spos], ssem).wait()

        @pl.when((nst == 0) & (sc >= 1))
        def _():
            pltpu.make_async_copy(stg, dst.at[spos], ssem).wait()


def _dot_body(u_ref, vi_ref, vj_ref, pi_ref, pj_ref):
    u = u_ref[:, :D]
    pi_ref[...] = jnp.sum(u * vi_ref[:, :D], axis=1)
    pj_ref[...] = jnp.sum(u * vj_ref[:, :D], axis=1)


def kernel(user, item_i, item_j, embed_user, embed_item):
    eu_t = embed_user.T
    ei_t = embed_item.T
    tail_u = jnp.pad(embed_user[PARTIAL_COL * 128:].T, ((0, 0), (0, D)))
    tail_i = jnp.pad(embed_item[PARTIAL_COL * 128:].T, ((0, 0), (0, D)))
    mesh = plsc.VectorSubcoreMesh(core_axis_name="c", subcore_axis_name="s")
    gather = pl.kernel(
        _gather_body,
        mesh=mesh,
        out_type=(jax.ShapeDtypeStruct((BROWS, 128), jnp.float32),
                  jax.ShapeDtypeStruct((BROWS, 128), jnp.float32),
                  jax.ShapeDtypeStruct((BROWS, 128), jnp.float32)),
        scratch_types=[
            pltpu.VMEM((B,), jnp.int32),
            pltpu.VMEM((WCAP,), jnp.int32),
            pltpu.VMEM((WCAP,), jnp.int32),
            pltpu.VMEM((WCAP,), jnp.int32),
            pltpu.VMEM((WCAP,), jnp.int32),
            pltpu.VMEM((WCAP,), jnp.int32),
            pltpu.VMEM((WCAP,), jnp.int32),
            pltpu.VMEM((WCAP,), jnp.int32),
            pltpu.VMEM((WCAP,), jnp.int32),
            pltpu.VMEM((D, CC * 128), jnp.float32),
            pltpu.VMEM((D, CC * 128), jnp.float32),
            pltpu.VMEM((L, 128), jnp.float32),
            pltpu.VMEM((L, 128), jnp.float32),
            pltpu.VMEM((L, 128), jnp.float32),
            pltpu.VMEM((L,), jnp.int32),
            pltpu.VMEM((L,), jnp.int32),
            pltpu.VMEM((L,), jnp.int32),
            pltpu.SemaphoreType.DMA,
            pltpu.SemaphoreType.DMA,
            pltpu.SemaphoreType.DMA,
            pltpu.SemaphoreType.DMA,
            pltpu.SemaphoreType.DMA,
        ],
        compiler_params=pltpu.CompilerParams(needs_layout_passes=False,
                                             use_tc_tiling_on_sc=True),
    )
    urows, virows, vjrows = gather(user, item_i, item_j, eu_t, ei_t,
                                   tail_u, tail_i)

    dot = pl.pallas_call(
        _dot_body,
        grid=(NW,),
        in_specs=[pl.BlockSpec((B // NW, 128), lambda i: (i, 0))] * 3,
        out_specs=[pl.BlockSpec((B // NW,), lambda i: (i,))] * 2,
        out_shape=(jax.ShapeDtypeStruct((B,), jnp.float32),
                   jax.ShapeDtypeStruct((B,), jnp.float32)),
    )
    return dot(urows, virows, vjrows)


# speedup vs baseline: 5.4675x; 1.0015x over previous
import jax
import jax.numpy as jnp
from jax import lax
from jax.experimental import pallas as pl
from jax.experimental.pallas import tpu as pltpu
from jax.experimental.pallas import tpu_sc as plsc

B = 16384
D = 64
E = 1000000

_info = plsc.get_sparse_core_info()
NC = _info.num_cores
NS = _info.num_subcores
L = _info.num_lanes
NW = NC * NS

TCOLS = 7813
PARTIAL_COL = 7812
CPW = 252
CC = 4
NCH = CPW // CC
WCAP = 1040
NVR = 1024
BROWS = B + L


def _popcount(mask):
    return plsc.all_reduce_population_count(mask)[0]


def _build_worklist(src_hbm, idxb, entl, posl, cbase):
    HB = B // 2

    def half(h, cnt0):
        pltpu.sync_copy(src_hbm.at[pl.ds(h * HB, HB)], idxb)

        def body(k, cnt):
            e = idxb[pl.ds(k * L, L)]
            ec = jax.lax.shift_right_logical(e, 7)
            m = (ec >= cbase) & (ec < cbase + CPW)
            plsc.store_compressed(entl.at[pl.ds(cnt, L)], e, mask=m)
            pos = h * HB + k * L + lax.iota(jnp.int32, L)
            plsc.store_compressed(posl.at[pl.ds(cnt, L)], pos, mask=m)
            return cnt + _popcount(m)
        return lax.fori_loop(0, HB // L, body, cnt0)
    return half(1, half(0, jnp.int32(0)))


def _chunk_dma(table, tail, slab, sem, cstart, start=True):
    @pl.when(cstart <= PARTIAL_COL - CC)
    def _():
        for di in range(D // 8):
            cp = pltpu.make_async_copy(
                table.at[pl.ds(di * 8, 8), pl.ds(cstart * 128, CC * 128)],
                slab.at[pl.ds(di * 8, 8), :], sem)
            cp.start() if start else cp.wait()

    @pl.when(cstart == PARTIAL_COL)
    def _():
        cp = pltpu.make_async_copy(tail, slab.at[:, pl.ds(0, 128)], sem)
        cp.start() if start else cp.wait()


def _process_chunk(slab, cstart, lists):
    new_sctrs = []
    for (entl, posl, cnt, dstrows, stg, spos, ssem, (sctr, nstg)) in lists:
        cel_m = jnp.full((L,), 0, jnp.int32)
        del cel_m

        def ext_body(k, carry):
            mcnt, = carry
            e = entl[pl.ds(k * L, L)]
            p = posl[pl.ds(k * L, L)]
            valid = (k * L + lax.iota(jnp.int32, L)) < cnt
            ecc = jax.lax.shift_right_logical(e, 7) - cstart
            m = valid & (ecc >= 0) & (ecc < CC)
            plsc.store_compressed(_process_chunk.ce.at[pl.ds(mcnt, L)], e, mask=m)
            plsc.store_compressed(_process_chunk.cp.at[pl.ds(mcnt, L)], p, mask=m)
            return (mcnt + _popcount(m),)

        nvreg = jax.lax.shift_right_logical(cnt + L - 1, 4)
        (mcnt,) = lax.fori_loop(0, nvreg, ext_body, (jnp.int32(0),))

        lanes = lax.iota(jnp.int32, L)

        def ent_body(n, carry):
            sc, nst = carry

            @pl.when((nst == 0) & (sc >= 1))
            def _():
                pltpu.make_async_copy(stg, dstrows.at[spos], ssem).wait()
            e_s = _process_chunk.ce[pl.ds(n, L)][0]
            p_s = _process_chunk.cp[pl.ds(n, L)][0]
            colin = jax.lax.shift_right_logical(e_s, 7) - cstart
            off = colin * 128 + jnp.bitwise_and(e_s, 127)
            offv = jnp.full((L,), off, jnp.int32)
            rowv = jnp.full((L,), nst, jnp.int32)
            for g in range(D // L):
                fv = g * L + lanes
                v = plsc.load_gather(slab, [fv, offv])
                plsc.store_scatter(stg, [rowv, fv], v)
            plsc.store_scatter(spos, [lanes], jnp.full((L,), p_s, jnp.int32),
                               mask=lanes == nst)

            @pl.when(nst == L - 1)
            def _():
                pltpu.make_async_copy(stg, dstrows.at[spos], ssem).start()
            full = nst == L - 1
            return (jnp.where(full, sc + 1, sc),
                    jnp.where(full, 0, nst + 1))

        new_sctrs.append(lax.fori_loop(0, mcnt, ent_body, (sctr, nstg)))
    return new_sctrs


def _ring_prologue(table, tail, slabs, dsems, cbase):
    for r in range(3):
        _chunk_dma(table, tail, slabs[r], dsems[r], cbase + r * CC, start=True)


def _table_scan(table, tail, slabs, dsems, cbase, lists, init_carries):
    def super_body(k, carries):
        for r in range(3):
            cid = 3 * k + r
            cstart = cbase + cid * CC
            _chunk_dma(table, tail, slabs[r], dsems[r], cstart, start=False)
            carries = tuple(_process_chunk(slabs[r], cstart,
                                           _bind(lists, carries)))

            @pl.when(cid + 3 < NCH)
            def _(r=r, cid=cid):
                _chunk_dma(table, tail, slabs[r], dsems[r],
                           cbase + (cid + 3) * CC, start=True)
        return carries

    return lax.fori_loop(0, NCH // 3, super_body,
                         tuple(carries for carries in init_carries))


def _bind(lists, sctrs):
    return [tuple(l) + (s,) for l, s in zip(lists, sctrs)]


def _gather_body(user, itemi, itemj, eu_t, ei_t, tail_u, tail_i,
                 urows, virows, vjrows,
                 idxb, ent_u, pos_u, ent_i, pos_i, ent_j, pos_j,
                 ce, cp, slabA, slabB, slabC,
                 stg_u, stg_i, stg_j, spos_u, spos_i, spos_j,
                 dsemA, dsemB, dsemC, ssem_u, ssem_i, ssem_j):
    _process_chunk.ce = ce
    _process_chunk.cp = cp
    wid = lax.axis_index("s") * NC + lax.axis_index("c")
    cbase = wid * CPW
    slabs = (slabA, slabB, slabC)
    dsems = (dsemA, dsemB, dsemC)

    _ring_prologue(eu_t, tail_u, slabs, dsems, cbase)
    cnt_u = _build_worklist(user, idxb, ent_u, pos_u, cbase)
    cnt_i = _build_worklist(itemi, idxb, ent_i, pos_i, cbase)
    cnt_j = _build_worklist(itemj, idxb, ent_j, pos_j, cbase)

    zz = (jnp.int32(0), jnp.int32(0))
    ((sc_u, nst_u),) = _table_scan(
        eu_t, tail_u, slabs, dsems, cbase,
        [(ent_u, pos_u, cnt_u, urows, stg_u, spos_u, ssem_u)], (zz,))
    _ring_prologue(ei_t, tail_i, slabs, dsems, cbase)
    ((sc_i, nst_i), (sc_j, nst_j)) = _table_scan(
        ei_t, tail_i, slabs, dsems, cbase,
        [(ent_i, pos_i, cnt_i, virows, stg_i, spos_i, ssem_i),
         (ent_j, pos_j, cnt_j, vjrows, stg_j, spos_j, ssem_j)], (zz, zz))

    lanes = lax.iota(jnp.int32, L)
    for sc, nst, stg, spos, dst, ssem in (
            (sc_u, nst_u, stg_u, spos_u, urows, ssem_u),
            (sc_i, nst_i, stg_i, spos_i, virows, ssem_i),
            (sc_j, nst_j, stg_j, spos_j, vjrows, ssem_j)):
        @pl.when(nst > 0)
        def _():
            plsc.store_scatter(spos, [lanes], jnp.full((L,), B, jnp.int32),
                               mask=lanes >= nst)
            pltpu.make_async_copy(stg, dst.at[spos], ssem).start()
            pltpu.make_async_copy(stg, dst.at[spos], ssem).wait()

        @pl.when((nst == 0) & (sc >= 1))
        def _():
            pltpu.make_async_copy(stg, dst.at[spos], ssem).wait()


def _dot_body(u_ref, vi_ref, vj_ref, pi_ref, pj_ref):
    u = u_ref[:, :D]
    pi_ref[...] = jnp.sum(u * vi_ref[:, :D], axis=1)
    pj_ref[...] = jnp.sum(u * vj_ref[:, :D], axis=1)


def kernel(user, item_i, item_j, embed_user, embed_item):
    eu_t = embed_user.T
    ei_t = embed_item.T
    tail_u = jnp.pad(embed_user[PARTIAL_COL * 128:].T, ((0, 0), (0, D)))
    tail_i = jnp.pad(embed_item[PARTIAL_COL * 128:].T, ((0, 0), (0, D)))
    mesh = plsc.VectorSubcoreMesh(core_axis_name="c", subcore_axis_name="s")
    gather = pl.kernel(
        _gather_body,
        mesh=mesh,
        out_type=(jax.ShapeDtypeStruct((BROWS, 128), jnp.float32),
                  jax.ShapeDtypeStruct((BROWS, 128), jnp.float32),
                  jax.ShapeDtypeStruct((BROWS, 128), jnp.float32)),
        scratch_types=[
            pltpu.VMEM((B // 2,), jnp.int32),
            pltpu.VMEM((WCAP,), jnp.int32),
            pltpu.VMEM((WCAP,), jnp.int32),
            pltpu.VMEM((WCAP,), jnp.int32),
            pltpu.VMEM((WCAP,), jnp.int32),
            pltpu.VMEM((WCAP,), jnp.int32),
            pltpu.VMEM((WCAP,), jnp.int32),
            pltpu.VMEM((WCAP,), jnp.int32),
            pltpu.VMEM((WCAP,), jnp.int32),
            pltpu.VMEM((D, CC * 128), jnp.float32),
            pltpu.VMEM((D, CC * 128), jnp.float32),
            pltpu.VMEM((D, CC * 128), jnp.float32),
            pltpu.VMEM((L, 128), jnp.float32),
            pltpu.VMEM((L, 128), jnp.float32),
            pltpu.VMEM((L, 128), jnp.float32),
            pltpu.VMEM((L,), jnp.int32),
            pltpu.VMEM((L,), jnp.int32),
            pltpu.VMEM((L,), jnp.int32),
            pltpu.SemaphoreType.DMA,
            pltpu.SemaphoreType.DMA,
            pltpu.SemaphoreType.DMA,
            pltpu.SemaphoreType.DMA,
            pltpu.SemaphoreType.DMA,
            pltpu.SemaphoreType.DMA,
        ],
        compiler_params=pltpu.CompilerParams(needs_layout_passes=False,
                                             use_tc_tiling_on_sc=True),
    )
    urows, virows, vjrows = gather(user, item_i, item_j, eu_t, ei_t,
                                   tail_u, tail_i)

    dot = pl.pallas_call(
        _dot_body,
        grid=(NW,),
        in_specs=[pl.BlockSpec((B // NW, 128), lambda i: (i, 0))] * 3,
        out_specs=[pl.BlockSpec((B // NW,), lambda i: (i,))] * 2,
        out_shape=(jax.ShapeDtypeStruct((B,), jnp.float32),
                   jax.ShapeDtypeStruct((B,), jnp.float32)),
    )
    return dot(urows, virows, vjrows)


# speedup vs baseline: 5.4804x; 1.0024x over previous
import jax
import jax.numpy as jnp
from jax import lax
from jax.experimental import pallas as pl
from jax.experimental.pallas import tpu as pltpu
from jax.experimental.pallas import tpu_sc as plsc

B = 16384
D = 64
E = 1000000

_info = plsc.get_sparse_core_info()
NC = _info.num_cores
NS = _info.num_subcores
L = _info.num_lanes
NW = NC * NS

TCOLS = 7813
PARTIAL_COL = 7812
CPW = 252
CC = 4
NCH = CPW // CC
WCAP = 1040
NVR = 1024
BROWS = B + L


def _popcount(mask):
    return plsc.all_reduce_population_count(mask)[0]


def _build_worklist(src_hbm, idxb, entl, posl, cbase):
    HB = B // 2

    def half(h, cnt0):
        pltpu.sync_copy(src_hbm.at[pl.ds(h * HB, HB)], idxb)

        def body(k, cnt):
            e = idxb[pl.ds(k * L, L)]
            ec = jax.lax.shift_right_logical(e, 7)
            m = (ec >= cbase) & (ec < cbase + CPW)
            plsc.store_compressed(entl.at[pl.ds(cnt, L)], e, mask=m)
            pos = h * HB + k * L + lax.iota(jnp.int32, L)
            plsc.store_compressed(posl.at[pl.ds(cnt, L)], pos, mask=m)
            return cnt + _popcount(m)
        return lax.fori_loop(0, HB // L, body, cnt0)
    return half(1, half(0, jnp.int32(0)))


def _chunk_dma(table, tail, slab, sem, cstart, start=True):
    @pl.when(cstart <= PARTIAL_COL - CC)
    def _():
        cp = pltpu.make_async_copy(
            table.at[pl.ds(0, D), pl.ds(cstart * 128, CC * 128)],
            slab, sem)
        cp.start() if start else cp.wait()

    @pl.when(cstart == PARTIAL_COL)
    def _():
        cp = pltpu.make_async_copy(tail, slab.at[:, pl.ds(0, 128)], sem)
        cp.start() if start else cp.wait()


def _process_chunk(slab, cstart, lists):
    new_sctrs = []
    for (entl, posl, cnt, dstrows, stg, spos, ssem, (sctr, nstg)) in lists:
        cel_m = jnp.full((L,), 0, jnp.int32)
        del cel_m

        def ext_body(k, carry):
            mcnt, = carry
            e = entl[pl.ds(k * L, L)]
            p = posl[pl.ds(k * L, L)]
            valid = (k * L + lax.iota(jnp.int32, L)) < cnt
            ecc = jax.lax.shift_right_logical(e, 7) - cstart
            m = valid & (ecc >= 0) & (ecc < CC)
            plsc.store_compressed(_process_chunk.ce.at[pl.ds(mcnt, L)], e, mask=m)
            plsc.store_compressed(_process_chunk.cp.at[pl.ds(mcnt, L)], p, mask=m)
            return (mcnt + _popcount(m),)

        nvreg = jax.lax.shift_right_logical(cnt + L - 1, 4)
        (mcnt,) = lax.fori_loop(0, nvreg, ext_body, (jnp.int32(0),))

        lanes = lax.iota(jnp.int32, L)

        def ent_body(n, carry):
            sc, nst = carry

            @pl.when((nst == 0) & (sc >= 1))
            def _():
                pltpu.make_async_copy(stg, dstrows.at[spos], ssem).wait()
            e_s = _process_chunk.ce[pl.ds(n, L)][0]
            p_s = _process_chunk.cp[pl.ds(n, L)][0]
            colin = jax.lax.shift_right_logical(e_s, 7) - cstart
            off = colin * 128 + jnp.bitwise_and(e_s, 127)
            offv = jnp.full((L,), off, jnp.int32)
            rowv = jnp.full((L,), nst, jnp.int32)
            for g in range(D // L):
                fv = g * L + lanes
                v = plsc.load_gather(slab, [fv, offv])
                plsc.store_scatter(stg, [rowv, fv], v)
            plsc.store_scatter(spos, [lanes], jnp.full((L,), p_s, jnp.int32),
                               mask=lanes == nst)

            @pl.when(nst == L - 1)
            def _():
                pltpu.make_async_copy(stg, dstrows.at[spos], ssem).start()
            full = nst == L - 1
            return (jnp.where(full, sc + 1, sc),
                    jnp.where(full, 0, nst + 1))

        new_sctrs.append(lax.fori_loop(0, mcnt, ent_body, (sctr, nstg)))
    return new_sctrs


def _ring_prologue(table, tail, slabs, dsems, cbase):
    for r in range(3):
        _chunk_dma(table, tail, slabs[r], dsems[r], cbase + r * CC, start=True)


def _table_scan(table, tail, slabs, dsems, cbase, lists, init_carries):
    def super_body(k, carries):
        for r in range(3):
            cid = 3 * k + r
            cstart = cbase + cid * CC
            _chunk_dma(table, tail, slabs[r], dsems[r], cstart, start=False)
            carries = tuple(_process_chunk(slabs[r], cstart,
                                           _bind(lists, carries)))

            @pl.when(cid + 3 < NCH)
            def _(r=r, cid=cid):
                _chunk_dma(table, tail, slabs[r], dsems[r],
                           cbase + (cid + 3) * CC, start=True)
        return carries

    return lax.fori_loop(0, NCH // 3, super_body,
                         tuple(carries for carries in init_carries))


def _bind(lists, sctrs):
    return [tuple(l) + (s,) for l, s in zip(lists, sctrs)]


def _gather_body(user, itemi, itemj, eu_t, ei_t, tail_u, tail_i,
                 urows, virows, vjrows,
                 idxb, ent_u, pos_u, ent_i, pos_i, ent_j, pos_j,
                 ce, cp, slabA, slabB, slabC,
                 stg_u, stg_i, stg_j, spos_u, spos_i, spos_j,
                 dsemA, dsemB, dsemC, ssem_u, ssem_i, ssem_j):
    _process_chunk.ce = ce
    _process_chunk.cp = cp
    wid = lax.axis_index("s") * NC + lax.axis_index("c")
    cbase = wid * CPW
    slabs = (slabA, slabB, slabC)
    dsems = (dsemA, dsemB, dsemC)

    _ring_prologue(eu_t, tail_u, slabs, dsems, cbase)
    cnt_u = _build_worklist(user, idxb, ent_u, pos_u, cbase)
    cnt_i = _build_worklist(itemi, idxb, ent_i, pos_i, cbase)
    cnt_j = _build_worklist(itemj, idxb, ent_j, pos_j, cbase)

    zz = (jnp.int32(0), jnp.int32(0))
    ((sc_u, nst_u),) = _table_scan(
        eu_t, tail_u, slabs, dsems, cbase,
        [(ent_u, pos_u, cnt_u, urows, stg_u, spos_u, ssem_u)], (zz,))
    _ring_prologue(ei_t, tail_i, slabs, dsems, cbase)
    ((sc_i, nst_i), (sc_j, nst_j)) = _table_scan(
        ei_t, tail_i, slabs, dsems, cbase,
        [(ent_i, pos_i, cnt_i, virows, stg_i, spos_i, ssem_i),
         (ent_j, pos_j, cnt_j, vjrows, stg_j, spos_j, ssem_j)], (zz, zz))

    lanes = lax.iota(jnp.int32, L)
    for sc, nst, stg, spos, dst, ssem in (
            (sc_u, nst_u, stg_u, spos_u, urows, ssem_u),
            (sc_i, nst_i, stg_i, spos_i, virows, ssem_i),
            (sc_j, nst_j, stg_j, spos_j, vjrows, ssem_j)):
        @pl.when(nst > 0)
        def _():
            plsc.store_scatter(spos, [lanes], jnp.full((L,), B, jnp.int32),
                               mask=lanes >= nst)
            pltpu.make_async_copy(stg, dst.at[spos], ssem).start()
            pltpu.make_async_copy(stg, dst.at[spos], ssem).wait()

        @pl.when((nst == 0) & (sc >= 1))
        def _():
            pltpu.make_async_copy(stg, dst.at[spos], ssem).wait()


def _dot_body(u_ref, vi_ref, vj_ref, pi_ref, pj_ref):
    u = u_ref[:, :D]
    pi_ref[...] = jnp.sum(u * vi_ref[:, :D], axis=1)
    pj_ref[...] = jnp.sum(u * vj_ref[:, :D], axis=1)


def kernel(user, item_i, item_j, embed_user, embed_item):
    eu_t = embed_user.T
    ei_t = embed_item.T
    tail_u = jnp.pad(embed_user[PARTIAL_COL * 128:].T, ((0, 0), (0, D)))
    tail_i = jnp.pad(embed_item[PARTIAL_COL * 128:].T, ((0, 0), (0, D)))
    mesh = plsc.VectorSubcoreMesh(core_axis_name="c", subcore_axis_name="s")
    gather = pl.kernel(
        _gather_body,
        mesh=mesh,
        out_type=(jax.ShapeDtypeStruct((BROWS, 128), jnp.float32),
                  jax.ShapeDtypeStruct((BROWS, 128), jnp.float32),
                  jax.ShapeDtypeStruct((BROWS, 128), jnp.float32)),
        scratch_types=[
            pltpu.VMEM((B // 2,), jnp.int32),
            pltpu.VMEM((WCAP,), jnp.int32),
            pltpu.VMEM((WCAP,), jnp.int32),
            pltpu.VMEM((WCAP,), jnp.int32),
            pltpu.VMEM((WCAP,), jnp.int32),
            pltpu.VMEM((WCAP,), jnp.int32),
            pltpu.VMEM((WCAP,), jnp.int32),
            pltpu.VMEM((WCAP,), jnp.int32),
            pltpu.VMEM((WCAP,), jnp.int32),
            pltpu.VMEM((D, CC * 128), jnp.float32),
            pltpu.VMEM((D, CC * 128), jnp.float32),
            pltpu.VMEM((D, CC * 128), jnp.float32),
            pltpu.VMEM((L, 128), jnp.float32),
            pltpu.VMEM((L, 128), jnp.float32),
            pltpu.VMEM((L, 128), jnp.float32),
            pltpu.VMEM((L,), jnp.int32),
            pltpu.VMEM((L,), jnp.int32),
            pltpu.VMEM((L,), jnp.int32),
            pltpu.SemaphoreType.DMA,
            pltpu.SemaphoreType.DMA,
            pltpu.SemaphoreType.DMA,
            pltpu.SemaphoreType.DMA,
            pltpu.SemaphoreType.DMA,
            pltpu.SemaphoreType.DMA,
        ],
        compiler_params=pltpu.CompilerParams(needs_layout_passes=False,
                                             use_tc_tiling_on_sc=True),
    )
    urows, virows, vjrows = gather(user, item_i, item_j, eu_t, ei_t,
                                   tail_u, tail_i)

    dot = pl.pallas_call(
        _dot_body,
        grid=(NW,),
        in_specs=[pl.BlockSpec((B // NW, 128), lambda i: (i, 0))] * 3,
        out_specs=[pl.BlockSpec((B // NW,), lambda i: (i,))] * 2,
        out_shape=(jax.ShapeDtypeStruct((B,), jnp.float32),
                   jax.ShapeDtypeStruct((B,), jnp.float32)),
    )
    return dot(urows, virows, vjrows)


# speedup vs baseline: 5.4927x; 1.0022x over previous
import jax
import jax.numpy as jnp
from jax import lax
from jax.experimental import pallas as pl
from jax.experimental.pallas import tpu as pltpu
from jax.experimental.pallas import tpu_sc as plsc

B = 16384
D = 64

_info = plsc.get_sparse_core_info()
NC = _info.num_cores
NS = _info.num_subcores
L = _info.num_lanes
NW = NC * NS

TCOLS = 7813
PARTIAL_COL = 7812
CPW = 252
CC = 4
NCH = CPW // CC
WCAP = 1040
BROWS = B + L


def _popcount(mask):
    return plsc.all_reduce_population_count(mask)[0]


def _build_worklist(src_hbm, idxb, entl, posl, cbase):
    HB = B // 2

    def half(h, cnt0):
        pltpu.sync_copy(src_hbm.at[pl.ds(h * HB, HB)], idxb)

        def body(k, cnt):
            e = idxb[pl.ds(k * L, L)]
            ec = jax.lax.shift_right_logical(e, 7)
            m = (ec >= cbase) & (ec < cbase + CPW)
            plsc.store_compressed(entl.at[pl.ds(cnt, L)], e, mask=m)
            pos = h * HB + k * L + lax.iota(jnp.int32, L)
            plsc.store_compressed(posl.at[pl.ds(cnt, L)], pos, mask=m)
            return cnt + _popcount(m)
        return lax.fori_loop(0, HB // L, body, cnt0)
    return half(1, half(0, jnp.int32(0)))


def _chunk_dma(table, tail, slab, sem, cstart, start=True):
    @pl.when(cstart <= PARTIAL_COL - CC)
    def _():
        cp = pltpu.make_async_copy(
            table.at[pl.ds(0, D), pl.ds(cstart * 128, CC * 128)],
            slab, sem)
        cp.start() if start else cp.wait()

    @pl.when(cstart == PARTIAL_COL)
    def _():
        cp = pltpu.make_async_copy(tail, slab.at[:, pl.ds(0, 128)], sem)
        cp.start() if start else cp.wait()


def _process_chunk(slab, cstart, lists):
    new_sctrs = []
    for (entl, posl, cnt, dstrows, stg, spos, ssem, (sctr, nstg)) in lists:
        def ext_body(k, carry):
            mcnt, = carry
            e = entl[pl.ds(k * L, L)]
            p = posl[pl.ds(k * L, L)]
            valid = (k * L + lax.iota(jnp.int32, L)) < cnt
            ecc = jax.lax.shift_right_logical(e, 7) - cstart
            m = valid & (ecc >= 0) & (ecc < CC)
            plsc.store_compressed(_process_chunk.ce.at[pl.ds(mcnt, L)], e, mask=m)
            plsc.store_compressed(_process_chunk.cp.at[pl.ds(mcnt, L)], p, mask=m)
            return (mcnt + _popcount(m),)

        nvreg = jax.lax.shift_right_logical(cnt + L - 1, 4)
        (mcnt,) = lax.fori_loop(0, nvreg, ext_body, (jnp.int32(0),))

        lanes = lax.iota(jnp.int32, L)

        def ent_body(n, carry):
            sc, nst = carry

            @pl.when((nst == 0) & (sc >= 1))
            def _():
                pltpu.make_async_copy(stg, dstrows.at[spos], ssem).wait()
            e_s = _process_chunk.ce[pl.ds(n, L)][0]
            p_s = _process_chunk.cp[pl.ds(n, L)][0]
            colin = jax.lax.shift_right_logical(e_s, 7) - cstart
            off = colin * 128 + jnp.bitwise_and(e_s, 127)
            offv = jnp.full((L,), off, jnp.int32)
            rowv = jnp.full((L,), nst, jnp.int32)
            for g in range(D // L):
                fv = g * L + lanes
                v = plsc.load_gather(slab, [fv, offv])
                plsc.store_scatter(stg, [rowv, fv], v)
            plsc.store_scatter(spos, [lanes], jnp.full((L,), p_s, jnp.int32),
                               mask=lanes == nst)

            @pl.when(nst == L - 1)
            def _():
                pltpu.make_async_copy(stg, dstrows.at[spos], ssem).start()
            full = nst == L - 1
            return (jnp.where(full, sc + 1, sc),
                    jnp.where(full, 0, nst + 1))

        new_sctrs.append(lax.fori_loop(0, mcnt, ent_body, (sctr, nstg)))
    return new_sctrs


def _ring_prologue(table, tail, slabs, dsems, cbase):
    for r in range(3):
        _chunk_dma(table, tail, slabs[r], dsems[r], cbase + r * CC, start=True)


def _table_scan(table, tail, slabs, dsems, cbase, lists, init_carries):
    def super_body(k, carries):
        for r in range(3):
            cid = 3 * k + r
            cstart = cbase + cid * CC
            _chunk_dma(table, tail, slabs[r], dsems[r], cstart, start=False)
            carries = tuple(_process_chunk(slabs[r], cstart,
                                           _bind(lists, carries)))

            @pl.when(cid + 3 < NCH)
            def _(r=r, cid=cid):
                _chunk_dma(table, tail, slabs[r], dsems[r],
                           cbase + (cid + 3) * CC, start=True)
        return carries

    return lax.fori_loop(0, NCH // 3, super_body,
                         tuple(carries for carries in init_carries))


def _bind(lists, sctrs):
    return [tuple(l) + (s,) for l, s in zip(lists, sctrs)]


def _gather_body(user, itemi, itemj, eu_t, ei_t, tail_u, tail_i,
                 urows, virows, vjrows,
                 idxb, ent_u, pos_u, ent_i, pos_i, ent_j, pos_j,
                 ce, cp, slabA, slabB, slabC,
                 stg_u, stg_i, stg_j, spos_u, spos_i, spos_j,
                 dsemA, dsemB, dsemC, ssem_u, ssem_i, ssem_j):
    _process_chunk.ce = ce
    _process_chunk.cp = cp
    wid = lax.axis_index("s") * NC + lax.axis_index("c")
    cbase = wid * CPW
    slabs = (slabA, slabB, slabC)
    dsems = (dsemA, dsemB, dsemC)

    _ring_prologue(eu_t, tail_u, slabs, dsems, cbase)
    cnt_u = _build_worklist(user, idxb, ent_u, pos_u, cbase)
    cnt_i = _build_worklist(itemi, idxb, ent_i, pos_i, cbase)
    cnt_j = _build_worklist(itemj, idxb, ent_j, pos_j, cbase)

    zz = (jnp.int32(0), jnp.int32(0))
    ((sc_u, nst_u),) = _table_scan(
        eu_t, tail_u, slabs, dsems, cbase,
        [(ent_u, pos_u, cnt_u, urows, stg_u, spos_u, ssem_u)], (zz,))
    _ring_prologue(ei_t, tail_i, slabs, dsems, cbase)
    ((sc_i, nst_i), (sc_j, nst_j)) = _table_scan(
        ei_t, tail_i, slabs, dsems, cbase,
        [(ent_i, pos_i, cnt_i, virows, stg_i, spos_i, ssem_i),
         (ent_j, pos_j, cnt_j, vjrows, stg_j, spos_j, ssem_j)], (zz, zz))

    lanes = lax.iota(jnp.int32, L)
    for sc, nst, stg, spos, dst, ssem in (
            (sc_u, nst_u, stg_u, spos_u, urows, ssem_u),
            (sc_i, nst_i, stg_i, spos_i, virows, ssem_i),
            (sc_j, nst_j, stg_j, spos_j, vjrows, ssem_j)):
        @pl.when(nst > 0)
        def _():
            plsc.store_scatter(spos, [lanes], jnp.full((L,), B, jnp.int32),
                               mask=lanes >= nst)
            pltpu.make_async_copy(stg, dst.at[spos], ssem).start()
            pltpu.make_async_copy(stg, dst.at[spos], ssem).wait()

        @pl.when((nst == 0) & (sc >= 1))
        def _():
            pltpu.make_async_copy(stg, dst.at[spos], ssem).wait()


def _dot_body(u_ref, vi_ref, vj_ref, pi_ref, pj_ref):
    u = u_ref[:, :D]
    pi_ref[...] = jnp.sum(u * vi_ref[:, :D], axis=1)
    pj_ref[...] = jnp.sum(u * vj_ref[:, :D], axis=1)


def kernel(user, item_i, item_j, embed_user, embed_item):
    eu_t = embed_user.T
    ei_t = embed_item.T
    tail_u = jnp.pad(embed_user[PARTIAL_COL * 128:].T, ((0, 0), (0, D)))
    tail_i = jnp.pad(embed_item[PARTIAL_COL * 128:].T, ((0, 0), (0, D)))
    mesh = plsc.VectorSubcoreMesh(core_axis_name="c", subcore_axis_name="s")
    gather = pl.kernel(
        _gather_body,
        mesh=mesh,
        out_type=(jax.ShapeDtypeStruct((BROWS, 128), jnp.float32),
                  jax.ShapeDtypeStruct((BROWS, 128), jnp.float32),
                  jax.ShapeDtypeStruct((BROWS, 128), jnp.float32)),
        scratch_types=[
            pltpu.VMEM((B // 2,), jnp.int32),
            pltpu.VMEM((WCAP,), jnp.int32),
            pltpu.VMEM((WCAP,), jnp.int32),
            pltpu.VMEM((WCAP,), jnp.int32),
            pltpu.VMEM((WCAP,), jnp.int32),
            pltpu.VMEM((WCAP,), jnp.int32),
            pltpu.VMEM((WCAP,), jnp.int32),
            pltpu.VMEM((WCAP,), jnp.int32),
            pltpu.VMEM((WCAP,), jnp.int32),
            pltpu.VMEM((D, CC * 128), jnp.float32),
            pltpu.VMEM((D, CC * 128), jnp.float32),
            pltpu.VMEM((D, CC * 128), jnp.float32),
            pltpu.VMEM((L, 128), jnp.float32),
            pltpu.VMEM((L, 128), jnp.float32),
            pltpu.VMEM((L, 128), jnp.float32),
            pltpu.VMEM((L,), jnp.int32),
            pltpu.VMEM((L,), jnp.int32),
            pltpu.VMEM((L,), jnp.int32),
            pltpu.SemaphoreType.DMA,
            pltpu.SemaphoreType.DMA,
            pltpu.SemaphoreType.DMA,
            pltpu.SemaphoreType.DMA,
            pltpu.SemaphoreType.DMA,
            pltpu.SemaphoreType.DMA,
        ],
        compiler_params=pltpu.CompilerParams(needs_layout_passes=False,
                                             use_tc_tiling_on_sc=True),
    )
    urows, virows, vjrows = gather(user, item_i, item_j, eu_t, ei_t,
                                   tail_u, tail_i)

    dot = pl.pallas_call(
        _dot_body,
        grid=(NW,),
        in_specs=[pl.BlockSpec((B // NW, 128), lambda i: (i, 0))] * 3,
        out_specs=[pl.BlockSpec((B // NW,), lambda i: (i,))] * 2,
        out_shape=(jax.ShapeDtypeStruct((B,), jnp.float32),
                   jax.ShapeDtypeStruct((B,), jnp.float32)),
    )
    return dot(urows, virows, vjrows)
